# baseline (speedup 1.0000x reference)
import functools

import jax
import jax.numpy as jnp
from jax import lax
from jax.experimental import pallas as pl
from jax.experimental.pallas import tpu as pltpu
from jax.experimental.pallas import tpu_sc as plsc

F, B, V, D = 26, 16384, 100000, 32
VP = 100096
NC, NS = 2, 16
FH = F // 2
W = 512
NCHUNK = 195
TAILW = 128
NR = F * VP * D // 128
NBT = B // 128
DT = D // 8


def _sc_body(x_hbm, tbl_hbm, bias_hbm, tail_hbm, out_hbm, scr_hbm,
             sbuf, tbuf_tail, dbuf, xb, qib, sub, qrows, tb, bb,
             asem, dsem, gsem):
    cid = lax.axis_index("c")
    sid = lax.axis_index("s")
    f0 = cid * FH
    iota = lax.iota(jnp.int32, 16)

    pltpu.sync_copy(bias_hbm, bb)

    def transpose_slab(buf, nv):
        @plsc.parallel_loop(0, nv, 1, unroll=4)
        def _t(vl):
            vv = jnp.full((16,), vl, jnp.int32)
            ca = plsc.load_gather(buf, [iota, vv])
            cb = plsc.load_gather(buf, [iota + 16, vv])
            r = vl >> 2
            l0 = (vl & 3) * 32
            dbuf[r, pl.ds(l0, 16)] = ca
            dbuf[r, pl.ds(l0 + 16, 16)] = cb

    def stage_slab(buf, f, v0, nv):
        for dt in range(DT):
            pltpu.async_copy(
                tbl_hbm.at[f, pl.ds(dt * 8, 8), pl.ds(v0, nv)],
                buf.at[pl.ds(dt * 8, 8)],
                asem,
            )
        for dt in range(DT):
            pltpu.make_async_copy(
                tbl_hbm.at[0, pl.ds(0, 8), pl.ds(0, nv)],
                buf.at[pl.ds(0, 8)],
                asem,
            ).wait()

    def write_dense(f, v0, nv):
        r0 = pl.multiple_of((f * VP + v0) // 4, 8)
        pltpu.async_copy(
            dbuf.at[pl.ds(0, nv // 4)], scr_hbm.at[pl.ds(r0, nv // 4)], dsem
        )
        pltpu.make_async_copy(
            dbuf.at[pl.ds(0, nv // 4)], scr_hbm.at[pl.ds(0, nv // 4)], dsem
        ).wait()

    for fi in range(FH):
        f = f0 + fi
        nchunks = 12 + jnp.where(sid < 3, 1, 0)

        def chunk_body(k, _, f=f):
            v0 = (sid + 16 * k) * W
            stage_slab(sbuf, f, v0, W)
            transpose_slab(sbuf, W)
            write_dense(f, v0, W)
            return 0

        lax.fori_loop(0, nchunks, chunk_body, 0)

    @pl.when(sid < FH)
    def _tail():
        f = f0 + sid
        stage_slab(tbuf_tail, f, NCHUNK * W, TAILW)
        transpose_slab(tbuf_tail, TAILW)
        write_dense(f, NCHUNK * W, TAILW)
        pltpu.async_copy(tail_hbm.at[f], dbuf.at[pl.ds(0, 8)], asem)
        pltpu.make_async_copy(
            tail_hbm.at[0], dbuf.at[pl.ds(0, 8)], asem
        ).wait()
        rT = pl.multiple_of((f * VP + NCHUNK * W + TAILW) // 4, 8)
        pltpu.async_copy(dbuf.at[pl.ds(0, 8)], scr_hbm.at[pl.ds(rT, 8)], dsem)
        pltpu.make_async_copy(
            dbuf.at[pl.ds(0, 8)], scr_hbm.at[pl.ds(0, 8)], dsem
        ).wait()

    plsc.subcore_barrier()

    def task_body(t, _):
        f = f0 + (t >> 4)
        btg = t & 15
        pltpu.sync_copy(x_hbm.at[pl.ds(f * B + btg * 1024, 1024)], xb)
        fvp = f * VP

        def qidx(g, _):
            v16 = xb[pl.ds(g * 16, 16)]
            qib[pl.ds(g * 16, 16)] = (v16 + fvp) >> 2
            sub[pl.ds(g * 16, 16)] = (v16 & 3) * 32
            return 0

        lax.fori_loop(0, 64, qidx, 0)
        bv0 = bb[f, pl.ds(0, 16)]
        bv1 = bb[f, pl.ds(16, 16)]

        def kb_body(kb, _, f=f, btg=btg):
            pltpu.async_copy(
                scr_hbm.at[qib.at[pl.ds(kb * 128, 128)]], qrows, gsem
            ).wait()
            for bg in range(8):
                rowi = iota + bg * 16
                base = sub[pl.ds(kb * 128 + bg * 16, 16)]
                for d in range(D):
                    val = plsc.load_gather(qrows, [rowi, base + d])
                    val = val + (bv0[d] if d < 16 else bv1[d - 16])
                    tb[d // 8, d % 8, pl.ds(bg * 16, 16)] = val
            for dt in range(DT):
                pltpu.sync_copy(tb.at[dt], out_hbm.at[f, dt, btg * 8 + kb])
            return 0

        lax.fori_loop(0, 8, kb_body, 0)
        return 0

    lax.fori_loop(0, FH * 16, task_body, 0)


@functools.partial(
    pl.kernel,
    out_type=(
        jax.ShapeDtypeStruct((F, DT, NBT, 8, 128), jnp.float32),
        jax.ShapeDtypeStruct((NR, 128), jnp.float32),
    ),
    mesh=plsc.VectorSubcoreMesh(core_axis_name="c", subcore_axis_name="s"),
    scratch_types=[
        pltpu.VMEM((32, W), jnp.float32),
        pltpu.VMEM((32, TAILW), jnp.float32),
        pltpu.VMEM((W // 4, 128), jnp.float32),
        pltpu.VMEM((1024,), jnp.int32),
        pltpu.VMEM((1024,), jnp.int32),
        pltpu.VMEM((1024,), jnp.int32),
        pltpu.VMEM((128, 128), jnp.float32),
        pltpu.VMEM((DT, 8, 128), jnp.float32),
        pltpu.VMEM((F, D), jnp.float32),
        pltpu.SemaphoreType.DMA,
        pltpu.SemaphoreType.DMA,
        pltpu.SemaphoreType.DMA,
    ],
    compiler_params=pltpu.CompilerParams(
        use_tc_tiling_on_sc=True, needs_layout_passes=False
    ),
)
def _tokenize_sc(x_hbm, tbl_hbm, bias_hbm, tail_hbm, out_hbm, scr_hbm, *rest):
    _sc_body(x_hbm, tbl_hbm, bias_hbm, tail_hbm, out_hbm, scr_hbm, *rest)


def kernel(x_dict, tables, bias):
    x1 = x_dict.astype(jnp.int32).reshape(F * B)
    tbl_t = jnp.transpose(tables, (0, 2, 1))
    tail26 = tables[:, V - 32:, :].reshape(F, 8, 128)
    out5, _ = _tokenize_sc(x1, tbl_t, bias, tail26)
    out = out5.transpose(0, 1, 3, 2, 4).reshape(F, D, B)
    return out.transpose(2, 0, 1)
